# baseline (device time: 18788 ns/iter reference)
import jax
import jax.numpy as jnp
from jax import lax
from jax.experimental import pallas as pl
from jax.experimental.pallas import tpu as pltpu

N_DEV = 4


def kernel(x, W1, W2):
    m, k = x.shape
    h_per = W1.shape[1]
    n = W2.shape[1]
    qh = m // N_DEV

    def body(x_ref, w1_ref, w2_ref, out_ref,
             send0_ref, recv0_ref, red_ref, recv1_ref,
             send0_sems, recv0_sems, send1_sems, recv1_sems):
        my_pos = lax.axis_index("i")

        barrier_sem = pltpu.get_barrier_semaphore()
        for k2 in range(1, N_DEV):
            pl.semaphore_signal(
                barrier_sem, inc=1,
                device_id=((my_pos + k2) % N_DEV,),
                device_id_type=pl.DeviceIdType.MESH,
            )
        pl.semaphore_wait(barrier_sem, N_DEV - 1)

        w1b = w1_ref[...].astype(jnp.bfloat16)
        w2b = w2_ref[...].astype(jnp.bfloat16)

        def quarter_partial(row_off):
            xc = x_ref[pl.ds(row_off, qh), :].astype(jnp.bfloat16)
            hc = jnp.maximum(
                jnp.dot(xc, w1b, preferred_element_type=jnp.float32), 0.0
            ).astype(jnp.bfloat16)
            return jnp.dot(hc, w2b, preferred_element_type=jnp.float32)

        rdma0 = []
        for k2 in range(1, N_DEV):
            target = (my_pos + k2) % N_DEV
            pc = quarter_partial(target * qh)
            send0_ref[k2 - 1] = pc.astype(jnp.bfloat16)
            r = pltpu.make_async_remote_copy(
                src_ref=send0_ref.at[k2 - 1],
                dst_ref=recv0_ref.at[k2 - 1],
                send_sem=send0_sems.at[k2 - 1],
                recv_sem=recv0_sems.at[k2 - 1],
                device_id=(target,),
                device_id_type=pl.DeviceIdType.MESH,
            )
            r.start()
            rdma0.append(r)

        acc = quarter_partial(my_pos * qh)
        for k2 in range(1, N_DEV):
            rdma0[k2 - 1].wait_recv()
            acc = acc + recv0_ref[k2 - 1].astype(jnp.float32)
        out_ref[pl.ds(my_pos * qh, qh), :] = acc
        red_ref[...] = acc.astype(jnp.bfloat16)

        rdma1 = []
        for k2 in range(1, N_DEV):
            r = pltpu.make_async_remote_copy(
                src_ref=red_ref,
                dst_ref=recv1_ref.at[k2 - 1],
                send_sem=send1_sems.at[k2 - 1],
                recv_sem=recv1_sems.at[k2 - 1],
                device_id=((my_pos + k2) % N_DEV,),
                device_id_type=pl.DeviceIdType.MESH,
            )
            r.start()
            rdma1.append(r)

        for k2 in range(1, N_DEV):
            rdma1[k2 - 1].wait_recv()
            src_pos = (my_pos - k2) % N_DEV
            out_ref[pl.ds(src_pos * qh, qh), :] = (
                recv1_ref[k2 - 1].astype(jnp.float32)
            )

        for r in rdma0 + rdma1:
            r.wait_send()

    return pl.pallas_call(
        body,
        out_shape=jax.ShapeDtypeStruct((m, n), jnp.float32),
        in_specs=[
            pl.BlockSpec(memory_space=pltpu.VMEM),
            pl.BlockSpec(memory_space=pltpu.VMEM),
            pl.BlockSpec(memory_space=pltpu.VMEM),
        ],
        out_specs=pl.BlockSpec(memory_space=pltpu.VMEM),
        scratch_shapes=[
            pltpu.VMEM((N_DEV - 1, qh, n), jnp.bfloat16),
            pltpu.VMEM((N_DEV - 1, qh, n), jnp.bfloat16),
            pltpu.VMEM((qh, n), jnp.bfloat16),
            pltpu.VMEM((N_DEV - 1, qh, n), jnp.bfloat16),
            pltpu.SemaphoreType.DMA((N_DEV - 1,)),
            pltpu.SemaphoreType.DMA((N_DEV - 1,)),
            pltpu.SemaphoreType.DMA((N_DEV - 1,)),
            pltpu.SemaphoreType.DMA((N_DEV - 1,)),
        ],
        compiler_params=pltpu.CompilerParams(collective_id=0),
    )(x, W1, W2)


# device time: 17307 ns/iter; 1.0856x vs baseline; 1.0856x over previous
import jax
import jax.numpy as jnp
from jax import lax
from jax.experimental import pallas as pl
from jax.experimental.pallas import tpu as pltpu

N_DEV = 4
N_CHUNK = 4


def kernel(x, W1, W2):
    m, k = x.shape
    h_per = W1.shape[1]
    n = W2.shape[1]

    def body(x_ref, w1_ref, w2_ref, out_ref, send_ref, recv_ref,
             send_sems, recv_sems):
        my_pos = lax.axis_index("i")
        left = (my_pos - 1) % N_DEV
        right = (my_pos + 1) % N_DEV

        barrier_sem = pltpu.get_barrier_semaphore()
        for nbr in (left, right):
            pl.semaphore_signal(
                barrier_sem, inc=1,
                device_id=(nbr,), device_id_type=pl.DeviceIdType.MESH,
            )
        pl.semaphore_wait(barrier_sem, 2)

        xb = x_ref[...].astype(jnp.bfloat16)
        w1b = w1_ref[...].astype(jnp.bfloat16)
        w2b = w2_ref[...].astype(jnp.bfloat16)

        partner_a = my_pos ^ 1
        partner_b = (N_DEV - 1) - my_pos
        cw = m // N_CHUNK

        def stage_partner(stage, c):
            if (c % 2 == 0) == (stage == 0):
                return partner_a
            return partner_b

        parts = []
        rdma_a = []
        for c in range(N_CHUNK):
            hc = jnp.maximum(
                jnp.dot(
                    xb[c * cw:(c + 1) * cw, :], w1b,
                    preferred_element_type=jnp.float32,
                ),
                0.0,
            ).astype(jnp.bfloat16)
            pc = jnp.dot(hc, w2b, preferred_element_type=jnp.float32)
            send_ref[c] = pc.astype(jnp.bfloat16)
            r = pltpu.make_async_remote_copy(
                src_ref=send_ref.at[c],
                dst_ref=recv_ref.at[c],
                send_sem=send_sems.at[c],
                recv_sem=recv_sems.at[c],
                device_id=(stage_partner(0, c),),
                device_id_type=pl.DeviceIdType.MESH,
            )
            r.start()
            parts.append(pc)
            rdma_a.append(r)

        accs = []
        rdma_b = []
        for c in range(N_CHUNK):
            rdma_a[c].wait_recv()
            acc = parts[c].astype(jnp.bfloat16) + recv_ref[c]
            send_ref[N_CHUNK + c] = acc
            r = pltpu.make_async_remote_copy(
                src_ref=send_ref.at[N_CHUNK + c],
                dst_ref=recv_ref.at[N_CHUNK + c],
                send_sem=send_sems.at[N_CHUNK + c],
                recv_sem=recv_sems.at[N_CHUNK + c],
                device_id=(stage_partner(1, c),),
                device_id_type=pl.DeviceIdType.MESH,
            )
            r.start()
            accs.append(acc)
            rdma_b.append(r)

        for c in range(N_CHUNK):
            rdma_b[c].wait_recv()
            out_ref[c * cw:(c + 1) * cw, :] = (
                accs[c] + recv_ref[N_CHUNK + c]
            ).astype(jnp.float32)

        for r in rdma_a + rdma_b:
            r.wait_send()

    return pl.pallas_call(
        body,
        out_shape=jax.ShapeDtypeStruct((m, n), jnp.float32),
        in_specs=[
            pl.BlockSpec(memory_space=pltpu.VMEM),
            pl.BlockSpec(memory_space=pltpu.VMEM),
            pl.BlockSpec(memory_space=pltpu.VMEM),
        ],
        out_specs=pl.BlockSpec(memory_space=pltpu.VMEM),
        scratch_shapes=[
            pltpu.VMEM((2 * N_CHUNK, m // N_CHUNK, n), jnp.bfloat16),
            pltpu.VMEM((2 * N_CHUNK, m // N_CHUNK, n), jnp.bfloat16),
            pltpu.SemaphoreType.DMA((2 * N_CHUNK,)),
            pltpu.SemaphoreType.DMA((2 * N_CHUNK,)),
        ],
        compiler_params=pltpu.CompilerParams(collective_id=0),
    )(x, W1, W2)
